# Initial kernel scaffold; baseline (speedup 1.0000x reference)
import jax
import jax.numpy as jnp
from jax.experimental import pallas as pl


def kernel(g, h, W, b):
    raise NotImplementedError("write your pallas kernel here")


# speedup vs baseline: 1.1234x; 1.1234x over previous
import functools

import jax
import jax.numpy as jnp
from jax import lax
from jax.experimental import pallas as pl
from jax.experimental.pallas import tpu as pltpu
from jax.experimental.pallas import tpu_sc as plsc

N = 4096
D = 256
K = 2048

_BI = 512
_BP = 512
_BM = 512


def _scores_body(h_ref, w_ref, b_ref, s_ref):
    w = jnp.dot(h_ref[...], w_ref[...], preferred_element_type=jnp.float32)
    s_ref[...] = jax.nn.sigmoid(w + b_ref[0, 0])


def _rank_body(s_col_ref, s_row_ref, r_ref):
    i = pl.program_id(0)
    s_i = s_col_ref[...]
    s_j = s_row_ref[...]
    jj = lax.broadcasted_iota(jnp.int32, (_BI, N), 1)
    ii = i * _BI + lax.broadcasted_iota(jnp.int32, (_BI, N), 0)
    ahead = (s_j > s_i) | ((s_j == s_i) & (jj < ii))
    r_ref[...] = jnp.sum(ahead.astype(jnp.int32), axis=1, keepdims=True)


def _inv_body(r_row_ref, s_row_ref, idx_ref, val_ref):
    p = pl.program_id(0)
    rr = r_row_ref[...]
    ss = s_row_ref[...]
    pp = p * _BI + lax.broadcasted_iota(jnp.int32, (_BI, N), 0)
    jj = lax.broadcasted_iota(jnp.int32, (_BI, N), 1)
    m = rr == pp
    idx_ref[...] = jnp.sum(jnp.where(m, jj, 0), axis=1, keepdims=True)
    val_ref[...] = jnp.sum(jnp.where(m, ss, 0.0), axis=1, keepdims=True)


def _prep_body(g_ref, u_ref, ut_ref):
    u = (g_ref[...] != 0).astype(jnp.float32)
    u_ref[...] = u
    ut_ref[...] = u.T


def _mm_body(r_ref, ct_ref, a_ref, deg_ref):
    j = pl.program_id(1)
    acc = lax.dot_general(r_ref[...].astype(jnp.bfloat16),
                          ct_ref[...].astype(jnp.bfloat16),
                          (((1,), (1,)), ((), ())),
                          preferred_element_type=jnp.float32)
    bin_f = (acc != 0).astype(jnp.float32)
    a_ref[...] = bin_f.astype(jnp.bfloat16)
    part = jnp.sum(bin_f, axis=1, keepdims=True)

    @pl.when(j == 0)
    def _():
        deg_ref[...] = part

    @pl.when(j != 0)
    def _():
        deg_ref[...] += part


def _fin_body(a_ref, deg_ref, hg_ref, val_ref, g_ref, h_ref):
    deg = deg_ref[...]
    deg = jnp.where(deg == 0, 1.0, deg)
    g_ref[...] = a_ref[...].astype(jnp.float32) / deg
    h_ref[...] = hg_ref[...] * val_ref[...]


def _make_sc_gather(nc, ns):
    nw = nc * ns
    rpw = K // nw
    chunk = rpw // 4
    mesh = plsc.VectorSubcoreMesh(core_axis_name="c", subcore_axis_name="s")

    @functools.partial(
        pl.kernel, mesh=mesh,
        out_type=[
            jax.ShapeDtypeStruct((K, N), jnp.float32),
            jax.ShapeDtypeStruct((K, N), jnp.float32),
            jax.ShapeDtypeStruct((K, D), jnp.float32),
        ],
        scratch_types=[
            pltpu.VMEM((rpw,), jnp.int32),
            pltpu.VMEM((chunk, N), jnp.float32),
            pltpu.VMEM((rpw, D), jnp.float32),
            pltpu.SemaphoreType.DMA,
        ],
    )
    def sc_gather(ug, ugt, h, idx, r_out, ct_out, hg_out, idx_v, buf, hbuf, sem):
        wid = lax.axis_index("s") * nc + lax.axis_index("c")
        base = wid * rpw
        pltpu.sync_copy(idx.at[pl.ds(base, rpw)], idx_v)
        pltpu.async_copy(h.at[idx_v], hbuf, sem).wait()
        pltpu.sync_copy(hbuf, hg_out.at[pl.ds(base, rpw)])
        for src, dst in ((ug, r_out), (ugt, ct_out)):
            for c in range(4):
                iv = idx_v.at[pl.ds(c * chunk, chunk)]
                pltpu.async_copy(src.at[iv], buf, sem).wait()
                pltpu.sync_copy(buf, dst.at[pl.ds(base + c * chunk, chunk)])

    return sc_gather


def kernel(g, h, W, b):
    f32 = jnp.float32
    s = pl.pallas_call(
        _scores_body,
        out_shape=jax.ShapeDtypeStruct((N, 1), f32),
    )(h, W, b.reshape(1, 1))
    s2 = s.reshape(1, N)

    r = pl.pallas_call(
        _rank_body,
        grid=(N // _BI,),
        in_specs=[
            pl.BlockSpec((_BI, 1), lambda i: (i, 0)),
            pl.BlockSpec((1, N), lambda i: (0, 0)),
        ],
        out_specs=pl.BlockSpec((_BI, 1), lambda i: (i, 0)),
        out_shape=jax.ShapeDtypeStruct((N, 1), jnp.int32),
    )(s, s2)
    r2 = r.reshape(1, N)

    idxf, vals = pl.pallas_call(
        _inv_body,
        grid=(N // _BI,),
        in_specs=[
            pl.BlockSpec((1, N), lambda i: (0, 0)),
            pl.BlockSpec((1, N), lambda i: (0, 0)),
        ],
        out_specs=[
            pl.BlockSpec((_BI, 1), lambda i: (i, 0)),
            pl.BlockSpec((_BI, 1), lambda i: (i, 0)),
        ],
        out_shape=[
            jax.ShapeDtypeStruct((N, 1), jnp.int32),
            jax.ShapeDtypeStruct((N, 1), f32),
        ],
    )(r2, s2)
    idx = idxf[:K, 0]
    values = vals[:K]

    ug, ugt = pl.pallas_call(
        _prep_body,
        grid=(N // _BP, N // _BP),
        in_specs=[pl.BlockSpec((_BP, _BP), lambda i, j: (i, j))],
        out_specs=[
            pl.BlockSpec((_BP, _BP), lambda i, j: (i, j)),
            pl.BlockSpec((_BP, _BP), lambda i, j: (j, i)),
        ],
        out_shape=[
            jax.ShapeDtypeStruct((N, N), jnp.float32),
            jax.ShapeDtypeStruct((N, N), jnp.float32),
        ],
    )(g)

    info = plsc.get_sparse_core_info()
    sc_gather = _make_sc_gather(info.num_cores, info.num_subcores)
    R, CT, Hg = sc_gather(ug, ugt, h, idx)

    a2b, deg = pl.pallas_call(
        _mm_body,
        grid=(K // _BM, K // _BM),
        in_specs=[
            pl.BlockSpec((_BM, N), lambda i, j: (i, 0)),
            pl.BlockSpec((_BM, N), lambda i, j: (j, 0)),
        ],
        out_specs=[
            pl.BlockSpec((_BM, _BM), lambda i, j: (i, j)),
            pl.BlockSpec((_BM, 1), lambda i, j: (i, 0)),
        ],
        out_shape=[
            jax.ShapeDtypeStruct((K, K), jnp.bfloat16),
            jax.ShapeDtypeStruct((K, 1), f32),
        ],
    )(R, CT)

    g_new, new_h = pl.pallas_call(
        _fin_body,
        grid=(K // _BM,),
        in_specs=[
            pl.BlockSpec((_BM, K), lambda i: (i, 0)),
            pl.BlockSpec((_BM, 1), lambda i: (i, 0)),
            pl.BlockSpec((_BM, D), lambda i: (i, 0)),
            pl.BlockSpec((_BM, 1), lambda i: (i, 0)),
        ],
        out_specs=[
            pl.BlockSpec((_BM, K), lambda i: (i, 0)),
            pl.BlockSpec((_BM, D), lambda i: (i, 0)),
        ],
        out_shape=[
            jax.ShapeDtypeStruct((K, K), f32),
            jax.ShapeDtypeStruct((K, D), f32),
        ],
    )(a2b, deg, Hg, values)

    return g_new, new_h, idx


# speedup vs baseline: 1.3616x; 1.2120x over previous
import functools

import jax
import jax.numpy as jnp
from jax import lax
from jax.experimental import pallas as pl
from jax.experimental.pallas import tpu as pltpu
from jax.experimental.pallas import tpu_sc as plsc

N = 4096
D = 256
K = 2048
H = N // 2

_BI = 512
_BP = 512
_BM = 512


def _scores_body(h_ref, w_ref, b_ref, s_ref):
    w = jnp.dot(h_ref[...], w_ref[...], preferred_element_type=jnp.float32)
    s_ref[...] = jax.nn.sigmoid(w + b_ref[0, 0])


def _rank_body(s_col_ref, s_row_ref, r_ref):
    i = pl.program_id(0)
    s_i = s_col_ref[...]
    s_j = s_row_ref[...]
    jj = lax.broadcasted_iota(jnp.int32, (_BI, N), 1)
    ii = i * _BI + lax.broadcasted_iota(jnp.int32, (_BI, N), 0)
    ahead = (s_j > s_i) | ((s_j == s_i) & (jj < ii))
    r_ref[...] = jnp.sum(ahead.astype(jnp.int32), axis=1, keepdims=True)


def _inv_body(r_row_ref, s_row_ref, idx_ref, val_ref):
    p = pl.program_id(0)
    rr = r_row_ref[...]
    ss = s_row_ref[...]
    pp = p * _BI + lax.broadcasted_iota(jnp.int32, (_BI, N), 0)
    jj = lax.broadcasted_iota(jnp.int32, (_BI, N), 1)
    m = rr == pp
    idx_ref[...] = jnp.sum(jnp.where(m, jj, 0), axis=1, keepdims=True)
    val_ref[...] = jnp.sum(jnp.where(m, ss, 0.0), axis=1, keepdims=True)


def _prep_body(ga_ref, gb_ref, gc_ref, gd_ref, up_ref, utp_ref):
    ua = (ga_ref[...] != 0).astype(jnp.int32)
    ub = (gb_ref[...] != 0).astype(jnp.int32)
    up_ref[...] = ua + 2 * ub
    uc = (gc_ref[...] != 0).astype(jnp.int32)
    ud = (gd_ref[...] != 0).astype(jnp.int32)
    utp_ref[...] = uc.T + 2 * ud.T


def _mm_body(r_ref, ct_ref, a_ref, deg_ref):
    j = pl.program_id(1)
    rp = r_ref[...]
    cp = ct_ref[...]
    rlo = (rp & 1).astype(jnp.bfloat16)
    rhi = (rp >> 1).astype(jnp.bfloat16)
    clo = (cp & 1).astype(jnp.bfloat16)
    chi = (cp >> 1).astype(jnp.bfloat16)
    dims = (((1,), (1,)), ((), ()))
    acc = lax.dot_general(rlo, clo, dims, preferred_element_type=jnp.float32)
    acc = acc + lax.dot_general(rhi, chi, dims,
                                preferred_element_type=jnp.float32)
    bin_f = (acc != 0).astype(jnp.float32)
    a_ref[...] = bin_f.astype(jnp.bfloat16)
    part = jnp.sum(bin_f, axis=1, keepdims=True)

    @pl.when(j == 0)
    def _():
        deg_ref[...] = part

    @pl.when(j != 0)
    def _():
        deg_ref[...] += part


def _fin_body(a_ref, deg_ref, hg_ref, val_ref, g_ref, h_ref):
    deg = deg_ref[...]
    deg = jnp.where(deg == 0, 1.0, deg)
    g_ref[...] = a_ref[...].astype(jnp.float32) / deg
    h_ref[...] = hg_ref[...] * val_ref[...]


def _make_sc_gather(nc, ns):
    nw = nc * ns
    rpw = K // nw
    chunk = rpw // 4
    mesh = plsc.VectorSubcoreMesh(core_axis_name="c", subcore_axis_name="s")

    @functools.partial(
        pl.kernel, mesh=mesh,
        out_type=[
            jax.ShapeDtypeStruct((K, H), jnp.int32),
            jax.ShapeDtypeStruct((K, H), jnp.int32),
            jax.ShapeDtypeStruct((K, D), jnp.float32),
        ],
        scratch_types=[
            pltpu.VMEM((rpw,), jnp.int32),
            pltpu.VMEM((chunk, H), jnp.int32),
            pltpu.VMEM((chunk, H), jnp.int32),
            pltpu.VMEM((rpw, D), jnp.float32),
            pltpu.SemaphoreType.DMA,
            pltpu.SemaphoreType.DMA,
            pltpu.SemaphoreType.DMA,
        ],
    )
    def sc_gather(up, utp, h, idx, rp_out, ctp_out, hg_out,
                  idx_v, buf0, buf1, hbuf, sem_g, sem_w0, sem_w1):
        wid = lax.axis_index("s") * nc + lax.axis_index("c")
        base = wid * rpw
        pltpu.sync_copy(idx.at[pl.ds(base, rpw)], idx_v)
        pltpu.async_copy(h.at[idx_v], hbuf, sem_g).wait()
        pltpu.sync_copy(hbuf, hg_out.at[pl.ds(base, rpw)])
        bufs = (buf0, buf1)
        wsems = (sem_w0, sem_w1)
        pending = [None, None]
        t = 0
        for src, dst in ((up, rp_out), (utp, ctp_out)):
            for c in range(4):
                sl = t % 2
                if pending[sl] is not None:
                    pending[sl].wait()
                iv = idx_v.at[pl.ds(c * chunk, chunk)]
                pltpu.async_copy(src.at[iv], bufs[sl], sem_g).wait()
                pending[sl] = pltpu.async_copy(
                    bufs[sl], dst.at[pl.ds(base + c * chunk, chunk)], wsems[sl])
                t += 1
        pending[0].wait()
        pending[1].wait()

    return sc_gather


def kernel(g, h, W, b):
    f32 = jnp.float32
    s = pl.pallas_call(
        _scores_body,
        out_shape=jax.ShapeDtypeStruct((N, 1), f32),
    )(h, W, b.reshape(1, 1))
    s2 = s.reshape(1, N)

    r = pl.pallas_call(
        _rank_body,
        grid=(N // _BI,),
        in_specs=[
            pl.BlockSpec((_BI, 1), lambda i: (i, 0)),
            pl.BlockSpec((1, N), lambda i: (0, 0)),
        ],
        out_specs=pl.BlockSpec((_BI, 1), lambda i: (i, 0)),
        out_shape=jax.ShapeDtypeStruct((N, 1), jnp.int32),
    )(s, s2)
    r2 = r.reshape(1, N)

    idxf, vals = pl.pallas_call(
        _inv_body,
        grid=(N // _BI,),
        in_specs=[
            pl.BlockSpec((1, N), lambda i: (0, 0)),
            pl.BlockSpec((1, N), lambda i: (0, 0)),
        ],
        out_specs=[
            pl.BlockSpec((_BI, 1), lambda i: (i, 0)),
            pl.BlockSpec((_BI, 1), lambda i: (i, 0)),
        ],
        out_shape=[
            jax.ShapeDtypeStruct((N, 1), jnp.int32),
            jax.ShapeDtypeStruct((N, 1), f32),
        ],
    )(r2, s2)
    idx = idxf[:K, 0]
    values = vals[:K]

    hb = H // _BP
    up, utp = pl.pallas_call(
        _prep_body,
        grid=(N // _BP, hb),
        in_specs=[
            pl.BlockSpec((_BP, _BP), lambda i, j: (i, j)),
            pl.BlockSpec((_BP, _BP), lambda i, j: (i, j + hb)),
            pl.BlockSpec((_BP, _BP), lambda i, j: (j, i)),
            pl.BlockSpec((_BP, _BP), lambda i, j: (j + hb, i)),
        ],
        out_specs=[
            pl.BlockSpec((_BP, _BP), lambda i, j: (i, j)),
            pl.BlockSpec((_BP, _BP), lambda i, j: (i, j)),
        ],
        out_shape=[
            jax.ShapeDtypeStruct((N, H), jnp.int32),
            jax.ShapeDtypeStruct((N, H), jnp.int32),
        ],
    )(g, g, g, g)

    info = plsc.get_sparse_core_info()
    sc_gather = _make_sc_gather(info.num_cores, info.num_subcores)
    Rp, CTp, Hg = sc_gather(up, utp, h, idx)

    a2b, deg = pl.pallas_call(
        _mm_body,
        grid=(K // _BM, K // _BM),
        in_specs=[
            pl.BlockSpec((_BM, H), lambda i, j: (i, 0)),
            pl.BlockSpec((_BM, H), lambda i, j: (j, 0)),
        ],
        out_specs=[
            pl.BlockSpec((_BM, _BM), lambda i, j: (i, j)),
            pl.BlockSpec((_BM, 1), lambda i, j: (i, 0)),
        ],
        out_shape=[
            jax.ShapeDtypeStruct((K, K), jnp.bfloat16),
            jax.ShapeDtypeStruct((K, 1), f32),
        ],
    )(Rp, CTp)

    g_new, new_h = pl.pallas_call(
        _fin_body,
        grid=(K // _BM,),
        in_specs=[
            pl.BlockSpec((_BM, K), lambda i: (i, 0)),
            pl.BlockSpec((_BM, 1), lambda i: (i, 0)),
            pl.BlockSpec((_BM, D), lambda i: (i, 0)),
            pl.BlockSpec((_BM, 1), lambda i: (i, 0)),
        ],
        out_specs=[
            pl.BlockSpec((_BM, K), lambda i: (i, 0)),
            pl.BlockSpec((_BM, D), lambda i: (i, 0)),
        ],
        out_shape=[
            jax.ShapeDtypeStruct((K, K), f32),
            jax.ShapeDtypeStruct((K, D), f32),
        ],
    )(a2b, deg, Hg, values)

    return g_new, new_h, idx
